# baseline (speedup 1.0000x reference)
import functools

import jax
import jax.numpy as jnp
from jax import lax
from jax.experimental import pallas as pl
from jax.experimental.pallas import tpu as pltpu
from jax.experimental.pallas import tpu_sc as plsc

TOPK = 16
BQ = 1024
TK = 2048
CH = 128
NCH_T = TK // CH

_SC_NC = 2
_SC_NS = 16
_NW = _SC_NC * _SC_NS
_G = 128

_IMAX = 2**31 - 1


def _p1_body(nkeys, qsq_ref, ksq_ref, q_ref, k_ref, d2_ref, cm_ref):
    kt = pl.program_id(1)
    q = q_ref[:]
    kb = k_ref[:]
    a = lax.dot_general(q, kb, (((1,), (1,)), ((), ())),
                        preferred_element_type=jnp.float32)
    d2 = (qsq_ref[:] - 2.0 * a) + ksq_ref[:]
    cols = kt * TK + lax.broadcasted_iota(jnp.int32, (BQ, TK), 1)
    d2 = jnp.where(cols < nkeys, d2, jnp.inf)
    mins = []
    for c in range(NCH_T):
        strip = d2[:, c * CH:(c + 1) * CH]
        d2_ref[c] = strip
        mins.append(jnp.min(strip, axis=1))
    cm_ref[0] = jnp.stack(mins, axis=1)


def _p2_body(nq, cm_ref, chunk_ref, rows_ref):
    qi = pl.program_id(0)
    cur = cm_ref[:]
    w = cur.shape[1]
    lane = lax.broadcasted_iota(jnp.int32, (BQ, w), 1)
    ids = []
    for _ in range(TOPK):
        m = jnp.min(cur, axis=1)
        pos = jnp.min(jnp.where(cur == m[:, None], lane, _IMAX), axis=1)
        ids.append(pos)
        cur = jnp.where(lane == pos[:, None], jnp.inf, cur)
    ch = jnp.stack(ids, axis=1)
    chunk_ref[:] = ch
    qg = qi * BQ + lax.broadcasted_iota(jnp.int32, (BQ, TOPK), 0)
    rows_ref[:] = ch * nq + qg


def _p3_body(gath_ref, ch_ref, dist_ref, idx_ref):
    cur = gath_ref[:]
    ch = ch_ref[:]
    lane = lax.broadcasted_iota(jnp.int32, (BQ, CH), 1)
    cols = jnp.concatenate(
        [jnp.broadcast_to(ch[:, c:c + 1] * CH, (BQ, CH)) + lane
         for c in range(TOPK)], axis=1)
    dvals, dids = [], []
    for _ in range(TOPK):
        m = jnp.min(cur, axis=1)
        eq = cur == m[:, None]
        cmin = jnp.min(jnp.where(eq, cols, _IMAX), axis=1)
        dvals.append(m)
        dids.append(cmin)
        cur = jnp.where(eq & (cols == cmin[:, None]), jnp.inf, cur)
    dist_ref[:] = jnp.stack(dvals, axis=1)
    idx_ref[:] = jnp.stack(dids, axis=1)


def _sc_gather_rows(table, rows2d):
    w = table.shape[1]
    ngrp_total = rows2d.shape[0]
    ng = ngrp_total // _NW
    mesh = plsc.VectorSubcoreMesh(core_axis_name="c", subcore_axis_name="s")

    @functools.partial(
        pl.kernel, mesh=mesh,
        out_type=jax.ShapeDtypeStruct((ngrp_total * _G, w), jnp.float32),
        scratch_types=[
            pltpu.VMEM((ng, _G), jnp.int32),
            pltpu.VMEM((_G, w), jnp.float32),
            pltpu.VMEM((_G, w), jnp.float32),
            pltpu.SemaphoreType.DMA,
            pltpu.SemaphoreType.DMA,
            pltpu.SemaphoreType.DMA,
        ],
    )
    def gk(table_hbm, rows_hbm, out_hbm, idx_all, buf0, buf1, gsem, w0, w1):
        wid = lax.axis_index("s") * _SC_NC + lax.axis_index("c")
        pltpu.sync_copy(rows_hbm.at[pl.ds(wid * ng, ng)], idx_all)
        bufs = (buf0, buf1)
        wsems = (w0, w1)
        pending = [None, None]
        for j in range(ng):
            b = j % 2
            if pending[b] is not None:
                pending[b].wait()
            pltpu.async_copy(table_hbm.at[idx_all.at[j]], bufs[b], gsem).wait()
            pending[b] = pltpu.async_copy(
                bufs[b], out_hbm.at[pl.ds((wid * ng + j) * _G, _G)], wsems[b])
        pending[0].wait()
        pending[1].wait()

    return gk(table, rows2d)


def kernel(queries, keys, k):
    del k
    nq, d = queries.shape
    nkeys = keys.shape[0]
    nkp = ((nkeys + TK - 1) // TK) * TK
    nt = nkp // TK
    nchunks = nkp // CH

    qsq = jnp.sum(queries * queries, axis=-1, keepdims=True)
    ksq = jnp.sum(keys * keys, axis=-1)[None, :]

    d2_full, cm = pl.pallas_call(
        functools.partial(_p1_body, nkeys),
        grid=(nq // BQ, nt),
        in_specs=[
            pl.BlockSpec((BQ, 1), lambda qi, ki: (qi, 0)),
            pl.BlockSpec((1, TK), lambda qi, ki: (0, ki)),
            pl.BlockSpec((BQ, d), lambda qi, ki: (qi, 0)),
            pl.BlockSpec((TK, d), lambda qi, ki: (ki, 0)),
        ],
        out_specs=[
            pl.BlockSpec((NCH_T, BQ, CH), lambda qi, ki: (ki, qi, 0)),
            pl.BlockSpec((1, BQ, NCH_T), lambda qi, ki: (ki, qi, 0)),
        ],
        out_shape=[
            jax.ShapeDtypeStruct((nchunks, nq, CH), jnp.float32),
            jax.ShapeDtypeStruct((nt, nq, NCH_T), jnp.float32),
        ],
        compiler_params=pltpu.CompilerParams(
            dimension_semantics=("parallel", "arbitrary")),
    )(qsq, ksq, queries, keys)

    cm2 = jnp.transpose(cm, (1, 0, 2)).reshape(nq, nchunks)
    wpad = ((nchunks + 127) // 128) * 128
    cm2 = jnp.pad(cm2, ((0, 0), (0, wpad - nchunks)),
                  constant_values=jnp.inf)

    chunk_ids, rows = pl.pallas_call(
        functools.partial(_p2_body, nq),
        grid=(nq // BQ,),
        in_specs=[pl.BlockSpec((BQ, wpad), lambda qi: (qi, 0))],
        out_specs=[
            pl.BlockSpec((BQ, TOPK), lambda qi: (qi, 0)),
            pl.BlockSpec((BQ, TOPK), lambda qi: (qi, 0)),
        ],
        out_shape=[
            jax.ShapeDtypeStruct((nq, TOPK), jnp.int32),
            jax.ShapeDtypeStruct((nq, TOPK), jnp.int32),
        ],
    )(cm2)

    strips = _sc_gather_rows(d2_full.reshape(nchunks * nq, CH),
                             rows.reshape(nq * TOPK // _G, _G))

    dists, idx = pl.pallas_call(
        _p3_body,
        grid=(nq // BQ,),
        in_specs=[
            pl.BlockSpec((BQ, TOPK * CH), lambda qi: (qi, 0)),
            pl.BlockSpec((BQ, TOPK), lambda qi: (qi, 0)),
        ],
        out_specs=[
            pl.BlockSpec((BQ, TOPK), lambda qi: (qi, 0)),
            pl.BlockSpec((BQ, TOPK), lambda qi: (qi, 0)),
        ],
        out_shape=[
            jax.ShapeDtypeStruct((nq, TOPK), jnp.float32),
            jax.ShapeDtypeStruct((nq, TOPK), jnp.int32),
        ],
    )(strips.reshape(nq, TOPK * CH), chunk_ids)

    keys_w = jnp.pad(keys, ((0, 0), (0, CH - d)))
    feats = _sc_gather_rows(keys_w, idx.reshape(nq * TOPK // _G, _G))
    neighbor_feats = feats[:, :d].reshape(nq, TOPK, d)
    return (dists, idx, neighbor_feats)


# speedup vs baseline: 1.0378x; 1.0378x over previous
import functools

import jax
import jax.numpy as jnp
from jax import lax
from jax.experimental import pallas as pl
from jax.experimental.pallas import tpu as pltpu
from jax.experimental.pallas import tpu_sc as plsc

TOPK = 16
BQ = 512
TK = 8192
CH = 128
NCH_T = TK // CH

_SC_NC = 2
_SC_NS = 16
_NW = _SC_NC * _SC_NS
_G = 128

_IMAX = 2**31 - 1


def _p1_body(nkeys, qsq_ref, ksq_ref, q_ref, k_ref, d2_ref, cm_ref):
    kt = pl.program_id(1)
    q = q_ref[:]
    kb = k_ref[:]
    a = lax.dot_general(q, kb, (((1,), (1,)), ((), ())),
                        preferred_element_type=jnp.float32)
    d2 = (qsq_ref[:] - 2.0 * a) + ksq_ref[:]
    cols = kt * TK + lax.broadcasted_iota(jnp.int32, (BQ, TK), 1)
    d2 = jnp.where(cols < nkeys, d2, jnp.inf)
    mins = []
    for c in range(NCH_T):
        strip = d2[:, c * CH:(c + 1) * CH]
        d2_ref[c] = strip
        mins.append(jnp.min(strip, axis=1))
    cm_ref[0] = jnp.stack(mins, axis=1)


def _p2_body(nq, cm_ref, chunk_ref, rows_ref):
    qi = pl.program_id(0)
    cur = cm_ref[:]
    w = cur.shape[1]
    lane = lax.broadcasted_iota(jnp.int32, (BQ, w), 1)
    ids = []
    for _ in range(TOPK):
        m = jnp.min(cur, axis=1)
        pos = jnp.min(jnp.where(cur == m[:, None], lane, _IMAX), axis=1)
        ids.append(pos)
        cur = jnp.where(lane == pos[:, None], jnp.inf, cur)
    ch = jnp.stack(ids, axis=1)
    chunk_ref[:] = ch
    qg = qi * BQ + lax.broadcasted_iota(jnp.int32, (BQ, TOPK), 0)
    rows_ref[:] = ch * nq + qg


def _p3_body(gath_ref, ch_ref, dist_ref, idx_ref):
    cur = gath_ref[:]
    ch = ch_ref[:]
    lane = lax.broadcasted_iota(jnp.int32, (BQ, CH), 1)
    cols = jnp.concatenate(
        [jnp.broadcast_to(ch[:, c:c + 1] * CH, (BQ, CH)) + lane
         for c in range(TOPK)], axis=1)
    dvals, dids = [], []
    for _ in range(TOPK):
        m = jnp.min(cur, axis=1)
        eq = cur == m[:, None]
        cmin = jnp.min(jnp.where(eq, cols, _IMAX), axis=1)
        dvals.append(m)
        dids.append(cmin)
        cur = jnp.where(eq & (cols == cmin[:, None]), jnp.inf, cur)
    dist_ref[:] = jnp.stack(dvals, axis=1)
    idx_ref[:] = jnp.stack(dids, axis=1)


def _sc_gather_rows(table, rows2d):
    w = table.shape[1]
    ngrp_total = rows2d.shape[0]
    ng = ngrp_total // _NW
    mesh = plsc.VectorSubcoreMesh(core_axis_name="c", subcore_axis_name="s")

    @functools.partial(
        pl.kernel, mesh=mesh,
        out_type=jax.ShapeDtypeStruct((ngrp_total * _G, w), jnp.float32),
        scratch_types=[
            pltpu.VMEM((ng, _G), jnp.int32),
            pltpu.VMEM((_G, w), jnp.float32),
            pltpu.VMEM((_G, w), jnp.float32),
            pltpu.SemaphoreType.DMA,
            pltpu.SemaphoreType.DMA,
            pltpu.SemaphoreType.DMA,
        ],
    )
    def gk(table_hbm, rows_hbm, out_hbm, idx_all, buf0, buf1, gsem, w0, w1):
        wid = lax.axis_index("s") * _SC_NC + lax.axis_index("c")
        pltpu.sync_copy(rows_hbm.at[pl.ds(wid * ng, ng)], idx_all)
        bufs = (buf0, buf1)
        wsems = (w0, w1)
        pending = [None, None]
        for j in range(ng):
            b = j % 2
            if pending[b] is not None:
                pending[b].wait()
            pltpu.async_copy(table_hbm.at[idx_all.at[j]], bufs[b], gsem).wait()
            pending[b] = pltpu.async_copy(
                bufs[b], out_hbm.at[pl.ds((wid * ng + j) * _G, _G)], wsems[b])
        pending[0].wait()
        pending[1].wait()

    return gk(table, rows2d)


def kernel(queries, keys, k):
    del k
    nq, d = queries.shape
    nkeys = keys.shape[0]
    nkp = ((nkeys + TK - 1) // TK) * TK
    nt = nkp // TK
    nchunks = nkp // CH

    qsq = jnp.sum(queries * queries, axis=-1, keepdims=True)
    ksq = jnp.sum(keys * keys, axis=-1)[None, :]

    d2_full, cm = pl.pallas_call(
        functools.partial(_p1_body, nkeys),
        grid=(nq // BQ, nt),
        in_specs=[
            pl.BlockSpec((BQ, 1), lambda qi, ki: (qi, 0)),
            pl.BlockSpec((1, TK), lambda qi, ki: (0, ki)),
            pl.BlockSpec((BQ, d), lambda qi, ki: (qi, 0)),
            pl.BlockSpec((TK, d), lambda qi, ki: (ki, 0)),
        ],
        out_specs=[
            pl.BlockSpec((NCH_T, BQ, CH), lambda qi, ki: (ki, qi, 0)),
            pl.BlockSpec((1, BQ, NCH_T), lambda qi, ki: (ki, qi, 0)),
        ],
        out_shape=[
            jax.ShapeDtypeStruct((nchunks, nq, CH), jnp.float32),
            jax.ShapeDtypeStruct((nt, nq, NCH_T), jnp.float32),
        ],
        compiler_params=pltpu.CompilerParams(
            dimension_semantics=("parallel", "arbitrary")),
    )(qsq, ksq, queries, keys)

    cm2 = jnp.transpose(cm, (1, 0, 2)).reshape(nq, nchunks)
    wpad = ((nchunks + 127) // 128) * 128
    cm2 = jnp.pad(cm2, ((0, 0), (0, wpad - nchunks)),
                  constant_values=jnp.inf)

    chunk_ids, rows = pl.pallas_call(
        functools.partial(_p2_body, nq),
        grid=(nq // BQ,),
        in_specs=[pl.BlockSpec((BQ, wpad), lambda qi: (qi, 0))],
        out_specs=[
            pl.BlockSpec((BQ, TOPK), lambda qi: (qi, 0)),
            pl.BlockSpec((BQ, TOPK), lambda qi: (qi, 0)),
        ],
        out_shape=[
            jax.ShapeDtypeStruct((nq, TOPK), jnp.int32),
            jax.ShapeDtypeStruct((nq, TOPK), jnp.int32),
        ],
    )(cm2)

    strips = _sc_gather_rows(d2_full.reshape(nchunks * nq, CH),
                             rows.reshape(nq * TOPK // _G, _G))

    dists, idx = pl.pallas_call(
        _p3_body,
        grid=(nq // BQ,),
        in_specs=[
            pl.BlockSpec((BQ, TOPK * CH), lambda qi: (qi, 0)),
            pl.BlockSpec((BQ, TOPK), lambda qi: (qi, 0)),
        ],
        out_specs=[
            pl.BlockSpec((BQ, TOPK), lambda qi: (qi, 0)),
            pl.BlockSpec((BQ, TOPK), lambda qi: (qi, 0)),
        ],
        out_shape=[
            jax.ShapeDtypeStruct((nq, TOPK), jnp.float32),
            jax.ShapeDtypeStruct((nq, TOPK), jnp.int32),
        ],
    )(strips.reshape(nq, TOPK * CH), chunk_ids)

    keys_w = jnp.pad(keys, ((0, 0), (0, CH - d)))
    feats = _sc_gather_rows(keys_w, idx.reshape(nq * TOPK // _G, _G))
    neighbor_feats = feats[:, :d].reshape(nq, TOPK, d)
    return (dists, idx, neighbor_feats)
